# baseline (speedup 1.0000x reference)
import functools

import jax
import jax.numpy as jnp
from jax import lax
from jax.experimental import pallas as pl
from jax.experimental.pallas import tpu as pltpu
from jax.experimental.pallas import tpu_sc as plsc

B = 16384
F = 26
V = 100000
D = 16

NC = 2
NS = 16
NW = NC * NS

TOT = 2 * B
SPT = TOT // NW
C = 64
NCHUNK = SPT // C
GRP = C * F // 128
GRP_PAD = 16
NCHUNK_TOT = TOT // C


VT_FULL = V // 128
VT_TAIL = V - VT_FULL * 128
WU = 2
WIDE = 128 * WU
N_UNITS = F * (VT_FULL // WU)
N_SING = F


NBUF = 4
ROUNDS = (N_UNITS // NW + NBUF) // NBUF


def _relayout_sc_body(fac2d_hbm, tail_hbm, out_hbm, *rest):
    wid = lax.axis_index("s") * NC + lax.axis_index("c")
    lanes = jnp.arange(16, dtype=jnp.int32)
    bufs = rest[0:NBUF]
    obufs = rest[NBUF:2 * NBUF]
    bufS = rest[2 * NBUF]
    obufS = rest[2 * NBUF + 1]
    sis = rest[2 * NBUF + 2:3 * NBUF + 2]
    sos = rest[3 * NBUF + 2:4 * NBUF + 2]

    def unit_fv(i):
        u = jnp.minimum(wid + i * NW, N_UNITS - 1)
        return u // (VT_FULL // WU), (u % (VT_FULL // WU)) * WIDE

    def src_at(f, v0, w):
        return fac2d_hbm.at[pl.ds(f * D, D), pl.ds(v0, w)]

    def dst_at(f, v0, w):
        return out_hbm.at[pl.ds((f * V + v0) * D, w * D)]

    def transpose(buf, obuf, w):
        for g in range(w // 16):
            tgt = (g * 16 + lanes) * D
            for d in range(D):
                vec = buf[d, pl.ds(g * 16, 16)]
                plsc.store_scatter(obuf, [tgt + d], vec)

    for b in range(NBUF):
        f, v0 = unit_fv(b)
        pltpu.async_copy(src_at(f, v0, WIDE), bufs[b], sis[b])
        pltpu.async_copy(obufs[b], dst_at(f, v0, WIDE), sos[b])

    def round_body(j, _):
        for b in range(NBUF):
            i = j * NBUF + b
            f, v0 = unit_fv(i)
            pltpu.make_async_copy(obufs[b], dst_at(f, v0, WIDE),
                                  sos[b]).wait()
            pltpu.make_async_copy(src_at(f, v0, WIDE), bufs[b],
                                  sis[b]).wait()
            transpose(bufs[b], obufs[b], WIDE)
            pltpu.async_copy(obufs[b], dst_at(f, v0, WIDE), sos[b])
            fn, v0n = unit_fv(i + NBUF)
            pltpu.async_copy(src_at(fn, v0n, WIDE), bufs[b], sis[b])
        return ()

    lax.fori_loop(0, ROUNDS, round_body, ())

    for b in range(NBUF):
        i = ROUNDS * NBUF + b
        f, v0 = unit_fv(i)
        pltpu.make_async_copy(obufs[b], dst_at(f, v0, WIDE), sos[b]).wait()
        pltpu.make_async_copy(src_at(f, v0, WIDE), bufs[b], sis[b]).wait()

    @pl.when(wid < F)
    def _():
        for vt in range(VT_FULL - WU * (VT_FULL // WU)):
            v0s = (WU * (VT_FULL // WU) + vt) * 128
            pltpu.sync_copy(src_at(wid, v0s, 128), bufS)
            transpose(bufS, obufS, 128)
            pltpu.sync_copy(obufS, dst_at(wid, v0s, 128))
        pltpu.sync_copy(
            tail_hbm.at[pl.ds(wid * VT_TAIL * D, VT_TAIL * D)],
            out_hbm.at[pl.ds((wid * V + VT_FULL * 128) * D, VT_TAIL * D)])


@functools.partial(
    pl.kernel,
    out_type=jax.ShapeDtypeStruct((F * V * D,), jnp.float32),
    mesh=plsc.VectorSubcoreMesh(core_axis_name="c", subcore_axis_name="s"),
    compiler_params=pltpu.CompilerParams(
        needs_layout_passes=False, use_tc_tiling_on_sc=True),
    scratch_types=(
        [pltpu.VMEM((D, WIDE), jnp.float32)] * NBUF
        + [pltpu.VMEM((WIDE * D,), jnp.float32)] * NBUF
        + [pltpu.VMEM((D, 128), jnp.float32),
           pltpu.VMEM((128 * D,), jnp.float32)]
        + [pltpu.SemaphoreType.DMA] * (2 * NBUF)
    ),
)
def _relayout_sc(*refs):
    _relayout_sc_body(*refs)


def _fm_body(idx_hbm, fac_hbm, lin_hbm, out_hbm,
             idx0, idx1, rows0, rows1, linv0, linv1, s_v, out_v,
             semf0, seml0, semf1, seml1):
    wid = lax.axis_index("s") * NC + lax.axis_index("c")
    lanes = jnp.arange(16, dtype=jnp.int32)
    sets = ((idx0, rows0, linv0, semf0, seml0),
            (idx1, rows1, linv1, semf1, seml1))

    def gather_cps(st, g):
        idxv, rowsv, linv, semf, seml = st
        cps = []
        for j in range(GRP):
            cps.append(pltpu.make_async_copy(
                fac_hbm.at[idxv.at[j]], rowsv.at[pl.ds(j * 128, 128)],
                semf))
            cps.append(pltpu.make_async_copy(
                lin_hbm.at[idxv.at[j]], linv.at[pl.ds(j * 128, 128)],
                seml))
        return cps

    def fetch(st, g):
        gc = jnp.minimum(g, NCHUNK - 1)
        row0 = (wid * NCHUNK + gc) * GRP_PAD
        pltpu.sync_copy(idx_hbm.at[pl.ds(row0, GRP_PAD)], st[0])
        for cp in gather_cps(st, g):
            cp.start()

    def drain(st, g):
        for cp in gather_cps(st, g):
            cp.wait()

    def compute(st, g):
        _, rowsv, linv, _, _ = st
        gc = jnp.minimum(g, NCHUNK - 1)

        def sum_body(s, _):
            acc0 = rowsv[s * F]
            acc1 = rowsv[s * F + 1]
            for f in range(2, F, 2):
                acc0 = acc0 + rowsv[s * F + f]
                acc1 = acc1 + rowsv[s * F + f + 1]
            s_v[pl.ds(s * D, D)] = acc0 + acc1
            return ()

        lax.fori_loop(0, C, sum_body, (), unroll=2)

        def red_body(i, _):
            rows16 = i * 16 + lanes
            t_sum = jnp.zeros((16,), jnp.float32)
            t_sq = jnp.zeros((16,), jnp.float32)
            base_d = rows16 * D
            for d in range(D):
                v = plsc.load_gather(s_v, [base_d + d])
                t_sum = t_sum + v
                t_sq = t_sq + v * v
            linacc = jnp.zeros((16,), jnp.float32)
            pos0 = rows16 * F
            for f in range(F):
                linacc = linacc + plsc.load_gather(linv, [pos0 + f])
            out_v[pl.ds(i * 16, 16)] = (
                linacc + 0.5 * (t_sum * t_sum - t_sq))
            return ()

        lax.fori_loop(0, C // 16, red_body, ())
        pltpu.sync_copy(out_v,
                        out_hbm.at[pl.ds((wid * NCHUNK + gc) * C, C)])

    fetch(sets[0], jnp.int32(0))

    def pair_body(j, _):
        g0 = 2 * j
        drain(sets[0], g0)
        fetch(sets[1], g0 + 1)
        compute(sets[0], g0)
        drain(sets[1], g0 + 1)
        fetch(sets[0], g0 + 2)
        compute(sets[1], g0 + 1)
        return ()

    lax.fori_loop(0, NCHUNK // 2, pair_body, ())
    drain(sets[0], jnp.int32(NCHUNK))


@functools.partial(
    pl.kernel,
    out_type=jax.ShapeDtypeStruct((TOT,), jnp.float32),
    mesh=plsc.VectorSubcoreMesh(core_axis_name="c", subcore_axis_name="s"),
    compiler_params=pltpu.CompilerParams(
        needs_layout_passes=False, use_tc_tiling_on_sc=False),
    scratch_types=(
        [pltpu.VMEM((GRP_PAD, 128), jnp.int32)] * 2
        + [pltpu.VMEM((C * F, D), jnp.float32)] * 2
        + [pltpu.VMEM((C * F,), jnp.float32)] * 2
        + [pltpu.VMEM((C * D,), jnp.float32),
           pltpu.VMEM((C,), jnp.float32)]
        + [pltpu.SemaphoreType.DMA] * 4
    ),
)
def _fm_kernel(*refs):
    _fm_body(*refs)


def kernel(pos_batch, neg_batch, lin_tables, fac_tables):
    off = (jnp.arange(F, dtype=jnp.int32) * V)[None, :]
    idx = jnp.concatenate([pos_batch.astype(jnp.int32),
                           neg_batch.astype(jnp.int32)], axis=0) + off
    idx_rows = jnp.pad(
        idx.reshape(NCHUNK_TOT, GRP, 128),
        ((0, 0), (0, GRP_PAD - GRP), (0, 0)),
    ).reshape(NCHUNK_TOT * GRP_PAD, 128)
    fac_t = jnp.transpose(fac_tables, (0, 2, 1))
    fac2d = fac_t.reshape(F * D, V)
    tail = fac_tables[:, VT_FULL * 128:, :].reshape(F * VT_TAIL * D)
    fac_flat = _relayout_sc(fac2d, tail).reshape(F * V, D)
    lin_flat = lin_tables.reshape(F * V)
    out = _fm_kernel(idx_rows, fac_flat, lin_flat)
    return out[:B], out[B:]


# speedup vs baseline: 1.1790x; 1.1790x over previous
import functools

import jax
import jax.numpy as jnp
from jax import lax
from jax.experimental import pallas as pl
from jax.experimental.pallas import tpu as pltpu
from jax.experimental.pallas import tpu_sc as plsc

B = 16384
F = 26
V = 100000
D = 16

NC = 2
NS = 16
NW = NC * NS

TOT = 2 * B
SPT = TOT // NW
C = 64
NCHUNK = SPT // C
GRP = C * F // 128
GRP_PAD = 16
NCHUNK_TOT = TOT // C


VT_FULL = V // 128
VT_TAIL = V - VT_FULL * 128
WU = 1
WIDE = 128 * WU
N_UNITS = F * (VT_FULL // WU)


NBUF = 4
ROUNDS = (N_UNITS // NW + NBUF) // NBUF


def _relayout_sc_body(fac2d_hbm, tail_hbm, out_hbm, *rest):
    wid = lax.axis_index("s") * NC + lax.axis_index("c")
    lanes = jnp.arange(16, dtype=jnp.int32)
    bufs = rest[0:NBUF]
    obufs = rest[NBUF:2 * NBUF]
    bufS = rest[2 * NBUF]
    obufS = rest[2 * NBUF + 1]
    sis = rest[2 * NBUF + 2:3 * NBUF + 2]
    sos = rest[3 * NBUF + 2:4 * NBUF + 2]

    def unit_fv(i):
        u = jnp.minimum(wid + i * NW, N_UNITS - 1)
        return u // (VT_FULL // WU), (u % (VT_FULL // WU)) * WIDE

    def src_at(f, v0, w):
        return fac2d_hbm.at[pl.ds(f * D, D), pl.ds(v0, w)]

    def dst_at(f, v0, w):
        return out_hbm.at[pl.ds((f * V + v0) * D, w * D)]

    def transpose(buf, obuf, w):
        for g in range(w // 16):
            tgt = (g * 16 + lanes) * D
            for d in range(D):
                vec = buf[d, pl.ds(g * 16, 16)]
                plsc.store_scatter(obuf, [tgt + d], vec)

    for b in range(NBUF):
        f, v0 = unit_fv(b)
        pltpu.async_copy(src_at(f, v0, WIDE), bufs[b], sis[b])
        pltpu.async_copy(obufs[b], dst_at(f, v0, WIDE), sos[b])

    def round_body(j, _):
        for b in range(NBUF):
            i = j * NBUF + b
            f, v0 = unit_fv(i)
            pltpu.make_async_copy(obufs[b], dst_at(f, v0, WIDE),
                                  sos[b]).wait()
            pltpu.make_async_copy(src_at(f, v0, WIDE), bufs[b],
                                  sis[b]).wait()
            transpose(bufs[b], obufs[b], WIDE)
            pltpu.async_copy(obufs[b], dst_at(f, v0, WIDE), sos[b])
            fn, v0n = unit_fv(i + NBUF)
            pltpu.async_copy(src_at(fn, v0n, WIDE), bufs[b], sis[b])
        return ()

    lax.fori_loop(0, ROUNDS, round_body, ())

    for b in range(NBUF):
        i = ROUNDS * NBUF + b
        f, v0 = unit_fv(i)
        pltpu.make_async_copy(obufs[b], dst_at(f, v0, WIDE), sos[b]).wait()
        pltpu.make_async_copy(src_at(f, v0, WIDE), bufs[b], sis[b]).wait()

    @pl.when(wid < F)
    def _():
        for vt in range(VT_FULL - WU * (VT_FULL // WU)):
            v0s = (WU * (VT_FULL // WU) + vt) * 128
            pltpu.sync_copy(src_at(wid, v0s, 128), bufS)
            transpose(bufS, obufS, 128)
            pltpu.sync_copy(obufS, dst_at(wid, v0s, 128))
        pltpu.sync_copy(
            tail_hbm.at[pl.ds(wid * VT_TAIL * D, VT_TAIL * D)],
            out_hbm.at[pl.ds((wid * V + VT_FULL * 128) * D, VT_TAIL * D)])


@functools.partial(
    pl.kernel,
    out_type=jax.ShapeDtypeStruct((F * V * D,), jnp.float32),
    mesh=plsc.VectorSubcoreMesh(core_axis_name="c", subcore_axis_name="s"),
    compiler_params=pltpu.CompilerParams(
        needs_layout_passes=False, use_tc_tiling_on_sc=True),
    scratch_types=(
        [pltpu.VMEM((D, WIDE), jnp.float32)] * NBUF
        + [pltpu.VMEM((WIDE * D,), jnp.float32)] * NBUF
        + [pltpu.VMEM((D, 128), jnp.float32),
           pltpu.VMEM((128 * D,), jnp.float32)]
        + [pltpu.SemaphoreType.DMA] * (2 * NBUF)
    ),
)
def _relayout_sc(*refs):
    _relayout_sc_body(*refs)


def _fm_body(idx_hbm, fac_hbm, lin_hbm, out_hbm,
             idx0, idx1, rows0, rows1, linv0, linv1, s_v, out_v,
             semf0, seml0, semf1, seml1):
    wid = lax.axis_index("s") * NC + lax.axis_index("c")
    lanes = jnp.arange(16, dtype=jnp.int32)
    sets = ((idx0, rows0, linv0, semf0, seml0),
            (idx1, rows1, linv1, semf1, seml1))

    def gather_cps(st, g):
        idxv, rowsv, linv, semf, seml = st
        cps = []
        for j in range(GRP):
            cps.append(pltpu.make_async_copy(
                fac_hbm.at[idxv.at[j]], rowsv.at[pl.ds(j * 128, 128)],
                semf))
            cps.append(pltpu.make_async_copy(
                lin_hbm.at[idxv.at[j]], linv.at[pl.ds(j * 128, 128)],
                seml))
        return cps

    def fetch(st, g):
        gc = jnp.minimum(g, NCHUNK - 1)
        row0 = (wid * NCHUNK + gc) * GRP_PAD
        pltpu.sync_copy(idx_hbm.at[pl.ds(row0, GRP_PAD)], st[0])
        for cp in gather_cps(st, g):
            cp.start()

    def drain(st, g):
        for cp in gather_cps(st, g):
            cp.wait()

    def compute(st, g):
        _, rowsv, linv, _, _ = st
        gc = jnp.minimum(g, NCHUNK - 1)

        def sum_body(s, _):
            acc0 = rowsv[s * F]
            acc1 = rowsv[s * F + 1]
            for f in range(2, F, 2):
                acc0 = acc0 + rowsv[s * F + f]
                acc1 = acc1 + rowsv[s * F + f + 1]
            s_v[pl.ds(s * D, D)] = acc0 + acc1
            return ()

        lax.fori_loop(0, C, sum_body, (), unroll=2)

        def red_body(i, _):
            rows16 = i * 16 + lanes
            t_sum = jnp.zeros((16,), jnp.float32)
            t_sq = jnp.zeros((16,), jnp.float32)
            base_d = rows16 * D
            for d in range(D):
                v = plsc.load_gather(s_v, [base_d + d])
                t_sum = t_sum + v
                t_sq = t_sq + v * v
            linacc = jnp.zeros((16,), jnp.float32)
            pos0 = rows16 * F
            for f in range(F):
                linacc = linacc + plsc.load_gather(linv, [pos0 + f])
            out_v[pl.ds(i * 16, 16)] = (
                linacc + 0.5 * (t_sum * t_sum - t_sq))
            return ()

        lax.fori_loop(0, C // 16, red_body, ())
        pltpu.sync_copy(out_v,
                        out_hbm.at[pl.ds((wid * NCHUNK + gc) * C, C)])

    fetch(sets[0], jnp.int32(0))

    def pair_body(j, _):
        g0 = 2 * j
        drain(sets[0], g0)
        fetch(sets[1], g0 + 1)
        compute(sets[0], g0)
        drain(sets[1], g0 + 1)
        fetch(sets[0], g0 + 2)
        compute(sets[1], g0 + 1)
        return ()

    lax.fori_loop(0, NCHUNK // 2, pair_body, ())
    drain(sets[0], jnp.int32(NCHUNK))


@functools.partial(
    pl.kernel,
    out_type=jax.ShapeDtypeStruct((TOT,), jnp.float32),
    mesh=plsc.VectorSubcoreMesh(core_axis_name="c", subcore_axis_name="s"),
    compiler_params=pltpu.CompilerParams(
        needs_layout_passes=False, use_tc_tiling_on_sc=False),
    scratch_types=(
        [pltpu.VMEM((GRP_PAD, 128), jnp.int32)] * 2
        + [pltpu.VMEM((C * F, D), jnp.float32)] * 2
        + [pltpu.VMEM((C * F,), jnp.float32)] * 2
        + [pltpu.VMEM((C * D,), jnp.float32),
           pltpu.VMEM((C,), jnp.float32)]
        + [pltpu.SemaphoreType.DMA] * 4
    ),
)
def _fm_kernel(*refs):
    _fm_body(*refs)


def kernel(pos_batch, neg_batch, lin_tables, fac_tables):
    off = (jnp.arange(F, dtype=jnp.int32) * V)[None, :]
    idx = jnp.concatenate([pos_batch.astype(jnp.int32),
                           neg_batch.astype(jnp.int32)], axis=0) + off
    idx_rows = jnp.pad(
        idx.reshape(NCHUNK_TOT, GRP, 128),
        ((0, 0), (0, GRP_PAD - GRP), (0, 0)),
    ).reshape(NCHUNK_TOT * GRP_PAD, 128)
    fac_t = jnp.transpose(fac_tables, (0, 2, 1))
    fac2d = fac_t.reshape(F * D, V)
    tail = fac_tables[:, VT_FULL * 128:, :].reshape(F * VT_TAIL * D)
    fac_flat = _relayout_sc(fac2d, tail).reshape(F * V, D)
    lin_flat = lin_tables.reshape(F * V)
    out = _fm_kernel(idx_rows, fac_flat, lin_flat)
    return out[:B], out[B:]
